# Initial kernel scaffold; baseline (speedup 1.0000x reference)
import jax
import jax.numpy as jnp
from jax.experimental import pallas as pl


def kernel(prefix, embedding_table):
    raise NotImplementedError("write your pallas kernel here")


# speedup vs baseline: 1.1018x; 1.1018x over previous
import functools

import jax
import jax.numpy as jnp
from jax import lax
from jax.experimental import pallas as pl
from jax.experimental.pallas import tpu as pltpu
from jax.experimental.pallas import tpu_sc as plsc

B = 512
D = 114688
NCH = 32
DC = D // NCH
NC = 2
NS = 16
NW = NC * NS
BPW = B // NW


def _body(pf_hbm, tbl_hbm, out_hbm, idx_v, buf, sem):
    wid = lax.axis_index("s") * NC + lax.axis_index("c")
    base = wid * BPW
    pltpu.sync_copy(pf_hbm.at[pl.ds(base, BPW)], idx_v)
    src0 = idx_v[...] * NCH
    dst0 = (base + lax.iota(jnp.int32, BPW)) * NCH

    def step(c, carry):
        src = src0 + c
        dst = dst0 + c
        pltpu.async_copy(tbl_hbm.at[src], buf, sem).wait()
        pltpu.async_copy(buf, out_hbm.at[dst], sem).wait()
        return carry

    lax.fori_loop(0, NCH, step, 0)


_gather = pl.kernel(
    _body,
    out_type=jax.ShapeDtypeStruct((B * NCH, DC), jnp.float32),
    mesh=plsc.VectorSubcoreMesh(core_axis_name="c", subcore_axis_name="s"),
    scratch_types=[
        pltpu.VMEM((BPW,), jnp.int32),
        pltpu.VMEM((BPW, DC), jnp.float32),
        pltpu.SemaphoreType.DMA,
    ],
)


def kernel(prefix, embedding_table):
    pf = prefix.reshape(B).astype(jnp.int32)
    tbl = embedding_table.reshape(128 * NCH, DC)
    out = _gather(pf, tbl)
    return out.reshape(prefix.shape[0], prefix.shape[1], D)


# speedup vs baseline: 1.1466x; 1.0407x over previous
import functools

import jax
import jax.numpy as jnp
from jax import lax
from jax.experimental import pallas as pl
from jax.experimental.pallas import tpu as pltpu
from jax.experimental.pallas import tpu_sc as plsc

B = 512
D = 114688
NCH = 32
DC = D // NCH
NC = 2
NS = 16
NW = NC * NS
BPW = B // NW


def _body(pf_hbm, tbl_hbm, out_hbm, idx_v, buf_a, buf_b, gsa, gsb, ssa, ssb):
    wid = lax.axis_index("s") * NC + lax.axis_index("c")
    base = wid * BPW
    pltpu.sync_copy(pf_hbm.at[pl.ds(base, BPW)], idx_v)
    src0 = idx_v[...] * NCH
    dst0 = (base + lax.iota(jnp.int32, BPW)) * NCH

    def gath(c, buf, sem):
        return pltpu.make_async_copy(tbl_hbm.at[src0 + c], buf, sem)

    def scat(c, buf, sem):
        return pltpu.make_async_copy(buf, out_hbm.at[dst0 + c], sem)

    gath(0, buf_a, gsa).start()
    gath(1, buf_b, gsb).start()
    gath(0, buf_a, gsa).wait()
    scat(0, buf_a, ssa).start()
    gath(1, buf_b, gsb).wait()
    scat(1, buf_b, ssb).start()

    def step(i, carry):
        c0 = 2 * i + 2
        c1 = 2 * i + 3
        scat(c0 - 2, buf_a, ssa).wait()
        gath(c0, buf_a, gsa).start()
        gath(c0, buf_a, gsa).wait()
        scat(c0, buf_a, ssa).start()
        scat(c1 - 2, buf_b, ssb).wait()
        gath(c1, buf_b, gsb).start()
        gath(c1, buf_b, gsb).wait()
        scat(c1, buf_b, ssb).start()
        return carry

    lax.fori_loop(0, NCH // 2 - 1, step, 0)
    scat(NCH - 2, buf_a, ssa).wait()
    scat(NCH - 1, buf_b, ssb).wait()


_gather = pl.kernel(
    _body,
    out_type=jax.ShapeDtypeStruct((B * NCH, DC), jnp.float32),
    mesh=plsc.VectorSubcoreMesh(core_axis_name="c", subcore_axis_name="s"),
    scratch_types=[
        pltpu.VMEM((BPW,), jnp.int32),
        pltpu.VMEM((BPW, DC), jnp.float32),
        pltpu.VMEM((BPW, DC), jnp.float32),
        pltpu.SemaphoreType.DMA,
        pltpu.SemaphoreType.DMA,
        pltpu.SemaphoreType.DMA,
        pltpu.SemaphoreType.DMA,
    ],
)


def kernel(prefix, embedding_table):
    pf = prefix.reshape(B).astype(jnp.int32)
    tbl = embedding_table.reshape(128 * NCH, DC)
    out = _gather(pf, tbl)
    return out.reshape(prefix.shape[0], prefix.shape[1], D)


# speedup vs baseline: 2.5117x; 2.1906x over previous
import jax
import jax.numpy as jnp
from jax import lax
from jax.experimental import pallas as pl
from jax.experimental.pallas import tpu as pltpu
from jax.experimental.pallas import tpu_sc as plsc

B = 512
D = 114688
NCH = 32
DC = D // NCH
NC = 2
NS = 16
NW = NC * NS
BPW = B // NW


def _body(pf_hbm, tbl_hbm, out_hbm, idx_v, buf_a, buf_b, gsa, gsb, ssa, ssb):
    wid = lax.axis_index("s") * NC + lax.axis_index("c")
    base = wid * BPW
    pltpu.sync_copy(pf_hbm.at[pl.ds(base, BPW)], idx_v)
    src = idx_v[...]

    def gath(c, buf, sem):
        return pltpu.make_async_copy(
            tbl_hbm.at[src, pl.ds(c * DC, DC)], buf, sem)

    def scat(c, buf, sem):
        return pltpu.make_async_copy(
            buf, out_hbm.at[pl.ds(base, BPW), pl.ds(c * DC, DC)], sem)

    gath(0, buf_a, gsa).start()
    gath(1, buf_b, gsb).start()
    gath(0, buf_a, gsa).wait()
    scat(0, buf_a, ssa).start()
    gath(1, buf_b, gsb).wait()
    scat(1, buf_b, ssb).start()

    def step(i, carry):
        c0 = 2 * i + 2
        c1 = 2 * i + 3
        scat(c0 - 2, buf_a, ssa).wait()
        gath(c0, buf_a, gsa).start()
        gath(c0, buf_a, gsa).wait()
        scat(c0, buf_a, ssa).start()
        scat(c1 - 2, buf_b, ssb).wait()
        gath(c1, buf_b, gsb).start()
        gath(c1, buf_b, gsb).wait()
        scat(c1, buf_b, ssb).start()
        return carry

    lax.fori_loop(0, NCH // 2 - 1, step, 0)
    scat(NCH - 2, buf_a, ssa).wait()
    scat(NCH - 1, buf_b, ssb).wait()


_gather = pl.kernel(
    _body,
    out_type=jax.ShapeDtypeStruct((B, D), jnp.float32),
    mesh=plsc.VectorSubcoreMesh(core_axis_name="c", subcore_axis_name="s"),
    scratch_types=[
        pltpu.VMEM((BPW,), jnp.int32),
        pltpu.VMEM((BPW, DC), jnp.float32),
        pltpu.VMEM((BPW, DC), jnp.float32),
        pltpu.SemaphoreType.DMA,
        pltpu.SemaphoreType.DMA,
        pltpu.SemaphoreType.DMA,
        pltpu.SemaphoreType.DMA,
    ],
)


def kernel(prefix, embedding_table):
    pf = prefix.reshape(B).astype(jnp.int32)
    out = _gather(pf, embedding_table)
    return out.reshape(prefix.shape[0], prefix.shape[1], D)


# speedup vs baseline: 2.5646x; 1.0211x over previous
import jax
import jax.numpy as jnp
from jax import lax
from jax.experimental import pallas as pl
from jax.experimental.pallas import tpu as pltpu
from jax.experimental.pallas import tpu_sc as plsc

B = 512
D = 114688
V = 128
NCH = 64
NCHS = NCH // 2
DC = D // NCH
NS = 16
RPT = B // NS
VPT = V // NS


def _body(pf_hbm, tbl_hbm, out_hbm, idx_v, sp_a, sp_b, stg_a, stg_b, dsem):
    core = lax.axis_index("c")
    s = lax.axis_index("s")
    tout = s * RPT
    pltpu.sync_copy(pf_hbm.at[pl.ds(tout, RPT)], idx_v)
    regs = [idx_v[pl.ds(0, 16)], idx_v[pl.ds(16, 16)]]
    rows = [regs[g][j] for g in range(2) for j in range(16)]

    def col(c):
        return (core * NCHS + c) * DC

    def stage(c, buf, sem):
        return pltpu.make_async_copy(
            tbl_hbm.at[pl.ds(s * VPT, VPT), pl.ds(col(c), DC)],
            buf.at[pl.ds(s * VPT, VPT), :], sem)

    def distribute(c, buf):
        cps = [
            pltpu.make_async_copy(
                buf.at[rows[j]], out_hbm.at[tout + j, pl.ds(col(c), DC)],
                dsem)
            for j in range(RPT)
        ]
        for cp in cps:
            cp.start()
        for cp in cps:
            cp.wait()

    last = jnp.int32(NCHS - 1)
    stage(0, sp_a, stg_a).start()
    stage(1, sp_b, stg_b).start()
    stage(0, sp_a, stg_a).wait()
    plsc.subcore_barrier()
    distribute(0, sp_a)
    plsc.subcore_barrier()
    stage(2, sp_a, stg_a).start()

    def step(i, carry):
        c0 = 2 * i + 1
        c1 = 2 * i + 2
        stage(c0, sp_b, stg_b).wait()
        plsc.subcore_barrier()
        distribute(c0, sp_b)
        plsc.subcore_barrier()
        stage(c0 + 2, sp_b, stg_b).start()
        stage(c1, sp_a, stg_a).wait()
        plsc.subcore_barrier()
        distribute(c1, sp_a)
        plsc.subcore_barrier()
        stage(jnp.minimum(c1 + 2, last), sp_a, stg_a).start()
        return carry

    lax.fori_loop(0, NCHS // 2 - 1, step, 0)
    stage(last, sp_b, stg_b).wait()
    plsc.subcore_barrier()
    distribute(NCHS - 1, sp_b)
    stage(last, sp_a, stg_a).wait()


_gather = pl.kernel(
    _body,
    out_type=jax.ShapeDtypeStruct((B, D), jnp.float32),
    mesh=plsc.VectorSubcoreMesh(core_axis_name="c", subcore_axis_name="s"),
    scratch_types=[
        pltpu.VMEM((RPT,), jnp.int32),
        pltpu.VMEM_SHARED((V, DC), jnp.float32),
        pltpu.VMEM_SHARED((V, DC), jnp.float32),
        pltpu.SemaphoreType.DMA,
        pltpu.SemaphoreType.DMA,
        pltpu.SemaphoreType.DMA,
    ],
)


def kernel(prefix, embedding_table):
    pf = prefix.reshape(B).astype(jnp.int32)
    out = _gather(pf, embedding_table)
    return out.reshape(prefix.shape[0], prefix.shape[1], D)
